# Initial kernel scaffold; baseline (speedup 1.0000x reference)
import jax
import jax.numpy as jnp
from jax.experimental import pallas as pl


def kernel(x, edge_index1, e_id1, edge_index2, e_id2, edge_weight, W1, b1, W2, b2):
    raise NotImplementedError("write your pallas kernel here")


# speedup vs baseline: 21.8019x; 21.8019x over previous
import functools

import jax
import jax.numpy as jnp
from jax import lax
from jax.experimental import pallas as pl
from jax.experimental.pallas import tpu as pltpu
from jax.experimental.pallas import tpu_sc as plsc

N_SRC0, N_DST1, N_DST2 = 10000, 4000, 1000
E1, E2, E_TOT = 320000, 64000, 400000
F_IN, F_HID, F_OUT = 128, 16, 64

NC, NS = 2, 16
NW = NC * NS
L = 16

OFF_DS1 = 0
OFF_DD1 = 10112
OFF_DS2 = 14208
OFF_DD2 = 18304
DEG_TOT = 19328

SUB = 128
E1T = 10240
E2T = 2048
E1P = E1T * NW
E2P = E2T * NW
EW_PAD = E_TOT + 8
ZERO_ID = E_TOT
ONE_ID = E_TOT + 1

_mesh = plsc.VectorSubcoreMesh(core_axis_name="c", subcore_axis_name="s")
_sc_params = pltpu.CompilerParams(needs_layout_passes=False,
                                  use_tc_tiling_on_sc=False)


def _wid():
    return lax.axis_index("s") * NC + lax.axis_index("c")


def _deg_body(r1, c1, i1, r2, c2, i2, ew, zer,
              w1o, w2o, degp,
              deg_v, rbuf, cbuf, ibuf, wful, sem0, sem1):
    wid = _wid()
    pltpu.sync_copy(zer.at[pl.ds(0, DEG_TOT)], deg_v)
    sems = (sem0, sem1)

    def run(row, col, eid, wout, n_tile, off_r, off_c):
        base = wid * n_tile
        nsub = n_tile // SUB
        pltpu.sync_copy(row.at[pl.ds(base, n_tile)], rbuf.at[pl.ds(0, n_tile)])
        pltpu.sync_copy(col.at[pl.ds(base, n_tile)], cbuf.at[pl.ds(0, n_tile)])
        pltpu.sync_copy(eid.at[pl.ds(base, n_tile)], ibuf.at[pl.ds(0, n_tile)])

        def fire(s, sem):
            pltpu.async_copy(ew.at[ibuf.at[pl.ds(s * SUB, SUB)]],
                             wful.at[pl.ds(s * SUB, SUB)], sem)

        def drain(s, sem):
            pltpu.make_async_copy(ew.at[pl.ds(0, SUB)],
                                  wful.at[pl.ds(s * SUB, SUB)], sem).wait()

        def compute(s):
            @plsc.parallel_loop(0, SUB // L, 1)
            def grp(g):
                gb = s * SUB + g * L
                r16 = rbuf[pl.ds(gb, L)]
                c16 = cbuf[pl.ds(gb, L)]
                if off_r:
                    r16 = r16 + off_r
                c16 = c16 + off_c
                w16 = wful[pl.ds(gb, L)]
                plsc.addupdate_scatter(deg_v, [r16], w16)
                plsc.addupdate_scatter(deg_v, [c16], w16)

        fire(0, sems[0])

        def outer(k, carry):
            s = k * 2
            fire(s + 1, sems[1])
            drain(s, sems[0])
            compute(s)

            @pl.when(s + 2 < nsub)
            def _():
                fire(s + 2, sems[0])

            drain(s + 1, sems[1])
            compute(s + 1)
            return carry

        lax.fori_loop(0, nsub // 2, outer, 0)
        pltpu.sync_copy(wful.at[pl.ds(0, n_tile)], wout.at[pl.ds(base, n_tile)])

    run(r1, c1, i1, w1o, E1T, OFF_DS1, OFF_DD1)
    run(r2, c2, i2, w2o, E2T, OFF_DS2, OFF_DD2)
    pltpu.sync_copy(deg_v, degp.at[wid])


@jax.jit
def _deg_call(r1, c1, i1, r2, c2, i2, ew, zer):
    return pl.kernel(
        _deg_body,
        out_type=[
            jax.ShapeDtypeStruct((E1P,), jnp.float32),
            jax.ShapeDtypeStruct((E2P,), jnp.float32),
            jax.ShapeDtypeStruct((NW, DEG_TOT), jnp.float32),
        ],
        mesh=_mesh,
        compiler_params=_sc_params,
        scratch_types=[
            pltpu.VMEM((DEG_TOT,), jnp.float32),
            pltpu.VMEM((E1T,), jnp.int32),
            pltpu.VMEM((E1T,), jnp.int32),
            pltpu.VMEM((E1T,), jnp.int32),
            pltpu.VMEM((E1T,), jnp.float32),
            pltpu.SemaphoreType.DMA,
            pltpu.SemaphoreType.DMA,
        ],
    )(r1, c1, i1, r2, c2, i2, ew, zer)


def _edge_group(rbuf, cbuf, wbuf, ds_v, dd_v, agg_v, xsrc, rowidx, gb, g):
    iota = lax.iota(jnp.int32, L)
    r16 = rbuf[pl.ds(gb, L)]
    c16 = cbuf[pl.ds(gb, L)]
    w16 = wbuf[pl.ds(gb, L)]
    a = plsc.load_gather(ds_v, [r16])
    b = plsc.load_gather(dd_v, [c16])
    norm = a * w16 * b
    xi = rowidx(g, r16, iota)
    cb = c16 * F_HID
    xs = [plsc.load_gather(xsrc, [xi, jnp.full((L,), f, jnp.int32)])
          for f in range(F_HID)]
    for f in range(F_HID):
        plsc.addupdate_scatter(agg_v, [cb + f], xs[f] * norm)


def _agg1_body(row, col, w, dinv, xlin, zer,
               aggp,
               agg_v, ds_v, dd_v, rbuf, cbuf, wbuf, rows0, rows1, sem0, sem1):
    wid = _wid()
    pltpu.sync_copy(zer.at[pl.ds(0, N_DST1 * F_HID)], agg_v)
    pltpu.sync_copy(dinv.at[pl.ds(OFF_DS1, N_SRC0)], ds_v)
    pltpu.sync_copy(dinv.at[pl.ds(OFF_DD1, N_DST1)], dd_v)
    base = wid * E1T
    pltpu.sync_copy(row.at[pl.ds(base, E1T)], rbuf)
    pltpu.sync_copy(col.at[pl.ds(base, E1T)], cbuf)
    pltpu.sync_copy(w.at[pl.ds(base, E1T)], wbuf)
    nsub = E1T // SUB
    bufs = (rows0, rows1)
    sems = (sem0, sem1)

    def fire(s, b):
        pltpu.async_copy(xlin.at[rbuf.at[pl.ds(s * SUB, SUB)]],
                         bufs[b], sems[b])

    def drain(b):
        pltpu.make_async_copy(xlin.at[pl.ds(0, SUB)], bufs[b], sems[b]).wait()

    def compute(s, b):
        rows_v = bufs[b]

        @plsc.parallel_loop(0, SUB // L, 1)
        def grp(g):
            _edge_group(rbuf, cbuf, wbuf, ds_v, dd_v, agg_v, rows_v,
                        lambda g_, r16, iota: g_ * L + iota, s * SUB + g * L, g)

    fire(0, 0)

    def outer(k, carry):
        s = k * 2
        fire(s + 1, 1)
        drain(0)
        compute(s, 0)

        @pl.when(s + 2 < nsub)
        def _():
            fire(s + 2, 0)

        drain(1)
        compute(s + 1, 1)
        return carry

    lax.fori_loop(0, nsub // 2, outer, 0)
    pltpu.sync_copy(agg_v, aggp.at[wid])


@jax.jit
def _agg1_call(row, col, w, dinv, xlin, zer):
    return pl.kernel(
        _agg1_body,
        out_type=jax.ShapeDtypeStruct((NW, N_DST1 * F_HID), jnp.float32),
        mesh=_mesh,
        compiler_params=_sc_params,
        scratch_types=[
            pltpu.VMEM((N_DST1 * F_HID,), jnp.float32),
            pltpu.VMEM((N_SRC0,), jnp.float32),
            pltpu.VMEM((N_DST1,), jnp.float32),
            pltpu.VMEM((E1T,), jnp.int32),
            pltpu.VMEM((E1T,), jnp.int32),
            pltpu.VMEM((E1T,), jnp.float32),
            pltpu.VMEM((SUB, F_HID), jnp.float32),
            pltpu.VMEM((SUB, F_HID), jnp.float32),
            pltpu.SemaphoreType.DMA,
            pltpu.SemaphoreType.DMA,
        ],
    )(row, col, w, dinv, xlin, zer)


def _agg2_body(row, col, w, dinv, h2d, zer,
               aggp,
               agg_v, ds_v, dd_v, rbuf, cbuf, wbuf, h_v):
    wid = _wid()
    pltpu.sync_copy(zer.at[pl.ds(0, N_DST2 * F_HID)], agg_v)
    pltpu.sync_copy(dinv.at[pl.ds(OFF_DS2, N_DST1)], ds_v)
    pltpu.sync_copy(dinv.at[pl.ds(OFF_DD2, N_DST2)], dd_v)
    pltpu.sync_copy(h2d, h_v)
    base = wid * E2T
    pltpu.sync_copy(row.at[pl.ds(base, E2T)], rbuf)
    pltpu.sync_copy(col.at[pl.ds(base, E2T)], cbuf)
    pltpu.sync_copy(w.at[pl.ds(base, E2T)], wbuf)

    @plsc.parallel_loop(0, E2T // L, 1)
    def grp(g):
        _edge_group(rbuf, cbuf, wbuf, ds_v, dd_v, agg_v, h_v,
                    lambda g_, r16, iota: r16, g * L, g)

    pltpu.sync_copy(agg_v, aggp.at[wid])


@jax.jit
def _agg2_call(row, col, w, dinv, h2d, zer):
    return pl.kernel(
        _agg2_body,
        out_type=jax.ShapeDtypeStruct((NW, N_DST2 * F_HID), jnp.float32),
        mesh=_mesh,
        compiler_params=_sc_params,
        scratch_types=[
            pltpu.VMEM((N_DST2 * F_HID,), jnp.float32),
            pltpu.VMEM((N_DST1,), jnp.float32),
            pltpu.VMEM((N_DST2,), jnp.float32),
            pltpu.VMEM((E2T,), jnp.int32),
            pltpu.VMEM((E2T,), jnp.int32),
            pltpu.VMEM((E2T,), jnp.float32),
            pltpu.VMEM((N_DST1, F_HID), jnp.float32),
        ],
    )(row, col, w, dinv, h2d, zer)


def _tc_prep_body(x_ref, w1_ref, degp_ref, xlin_ref, dinv_ref):
    xlin_ref[...] = jnp.dot(x_ref[...], w1_ref[...],
                            preferred_element_type=jnp.float32)
    deg = jnp.sum(degp_ref[...], axis=0)
    dinv_ref[...] = jnp.where(deg > 0.0, lax.rsqrt(deg), 0.0)


@jax.jit
def _tc_prep(x, W1, degp):
    return pl.pallas_call(
        _tc_prep_body,
        out_shape=[
            jax.ShapeDtypeStruct((N_SRC0, F_HID), jnp.float32),
            jax.ShapeDtypeStruct((151, 128), jnp.float32),
        ],
    )(x, W1, degp)


def _tc_relu_body(aggp_ref, b1_ref, h_ref):
    s = jnp.sum(aggp_ref[...], axis=0)
    h_ref[...] = jnp.maximum(s + b1_ref[...], 0.0)


@jax.jit
def _tc_relu(aggp, b1t):
    return pl.pallas_call(
        _tc_relu_body,
        out_shape=jax.ShapeDtypeStruct((N_DST1 * F_HID // 128, 128),
                                       jnp.float32),
    )(aggp, b1t)


def _tc_final_body(aggp_ref, w2_ref, b2_ref, out_ref):
    agg = jnp.sum(aggp_ref[...], axis=0)
    o = jnp.dot(agg, w2_ref[...], preferred_element_type=jnp.float32)
    o = o + b2_ref[...]
    z = o - jnp.max(o, axis=1, keepdims=True)
    out_ref[...] = z - jnp.log(jnp.sum(jnp.exp(z), axis=1, keepdims=True))


@jax.jit
def _tc_final(aggp, W2, b2):
    return pl.pallas_call(
        _tc_final_body,
        out_shape=jax.ShapeDtypeStruct((N_DST2, F_OUT), jnp.float32),
    )(aggp, W2, b2)


def kernel(x, edge_index1, e_id1, edge_index2, e_id2, edge_weight,
           W1, b1, W2, b2):
    i32 = jnp.int32
    loops1 = jnp.arange(N_DST1, dtype=i32)
    loops2 = jnp.arange(N_DST2, dtype=i32)
    pad1 = E1P - E1 - N_DST1
    pad2 = E2P - E2 - N_DST2
    z1 = jnp.zeros((pad1,), i32)
    z2 = jnp.zeros((pad2,), i32)
    r1 = jnp.concatenate([edge_index1[0], loops1, z1])
    c1 = jnp.concatenate([edge_index1[1], loops1, z1])
    i1 = jnp.concatenate([e_id1, jnp.full((N_DST1,), ONE_ID, i32),
                          jnp.full((pad1,), ZERO_ID, i32)])
    r2 = jnp.concatenate([edge_index2[0], loops2, z2])
    c2 = jnp.concatenate([edge_index2[1], loops2, z2])
    i2 = jnp.concatenate([e_id2, jnp.full((N_DST2,), ONE_ID, i32),
                          jnp.full((pad2,), ZERO_ID, i32)])
    ew = jnp.concatenate([
        edge_weight,
        jnp.array([0.0, 1.0], jnp.float32),
        jnp.zeros((EW_PAD - E_TOT - 2,), jnp.float32),
    ])
    zer = jnp.zeros((N_DST1 * F_HID,), jnp.float32)

    w1, w2, degp = _deg_call(r1, c1, i1, r2, c2, i2, ew, zer)
    xlin, dinv2d = _tc_prep(x, W1, degp.reshape(NW, 151, 128))
    dinv = dinv2d.reshape(DEG_TOT)
    aggp1 = _agg1_call(r1, c1, w1, dinv, xlin, zer)
    b1t = jnp.tile(b1, 128 // F_HID).reshape(1, 128)
    h = _tc_relu(aggp1.reshape(NW, N_DST1 * F_HID // 128, 128), b1t)
    aggp2 = _agg2_call(r2, c2, w2, dinv, h.reshape(N_DST1, F_HID), zer)
    return _tc_final(aggp2.reshape(NW, N_DST2, F_HID), W2,
                     b2.reshape(1, F_OUT))


# speedup vs baseline: 22.1533x; 1.0161x over previous
import functools

import jax
import jax.numpy as jnp
from jax import lax
from jax.experimental import pallas as pl
from jax.experimental.pallas import tpu as pltpu
from jax.experimental.pallas import tpu_sc as plsc

N_SRC0, N_DST1, N_DST2 = 10000, 4000, 1000
E1, E2, E_TOT = 320000, 64000, 400000
F_IN, F_HID, F_OUT = 128, 16, 64

NC, NS = 2, 16
NW = NC * NS
L = 16

OFF_DS1 = 0
OFF_DD1 = 10112
OFF_DS2 = 14208
OFF_DD2 = 18304
DEG_TOT = 19328

SUB = 128
E1T = 10240
E2T = 2048
E1P = E1T * NW
E2P = E2T * NW
EW_PAD = E_TOT + 8
ZERO_ID = E_TOT
ONE_ID = E_TOT + 1

_mesh = plsc.VectorSubcoreMesh(core_axis_name="c", subcore_axis_name="s")
_sc_params = pltpu.CompilerParams(needs_layout_passes=False,
                                  use_tc_tiling_on_sc=False)


def _wid():
    return lax.axis_index("s") * NC + lax.axis_index("c")


def _deg_body(r1, c1, i1, r2, c2, i2, ew, zer,
              w1o, w2o, degp,
              deg_v, rbuf, cbuf, ibuf, wful, sem0, sem1, sem2, sem3):
    wid = _wid()
    pltpu.sync_copy(zer.at[pl.ds(0, DEG_TOT)], deg_v)
    sems = (sem0, sem1, sem2, sem3)

    def run(row, col, eid, wout, n_tile, off_r, off_c):
        base = wid * n_tile
        nsub = n_tile // SUB
        pltpu.sync_copy(row.at[pl.ds(base, n_tile)], rbuf.at[pl.ds(0, n_tile)])
        pltpu.sync_copy(col.at[pl.ds(base, n_tile)], cbuf.at[pl.ds(0, n_tile)])
        pltpu.sync_copy(eid.at[pl.ds(base, n_tile)], ibuf.at[pl.ds(0, n_tile)])

        def fire(s, sem):
            pltpu.async_copy(ew.at[ibuf.at[pl.ds(s * SUB, SUB)]],
                             wful.at[pl.ds(s * SUB, SUB)], sem)

        def drain(s, sem):
            pltpu.make_async_copy(ew.at[pl.ds(0, SUB)],
                                  wful.at[pl.ds(s * SUB, SUB)], sem).wait()

        def compute(s):
            @plsc.parallel_loop(0, SUB // L, 1)
            def grp(g):
                gb = s * SUB + g * L
                r16 = rbuf[pl.ds(gb, L)]
                c16 = cbuf[pl.ds(gb, L)]
                if off_r:
                    r16 = r16 + off_r
                c16 = c16 + off_c
                w16 = wful[pl.ds(gb, L)]
                plsc.addupdate_scatter(deg_v, [r16], w16)
                plsc.addupdate_scatter(deg_v, [c16], w16)

        for b in range(3):
            fire(b, sems[b])

        def outer(k, carry):
            s = k * 4
            for b in range(4):
                sb = s + b

                @pl.when(sb + 3 < nsub)
                def _():
                    fire(sb + 3, sems[(b + 3) % 4])

                drain(sb, sems[b])
                compute(sb)
            return carry

        lax.fori_loop(0, nsub // 4, outer, 0)
        pltpu.sync_copy(wful.at[pl.ds(0, n_tile)], wout.at[pl.ds(base, n_tile)])

    run(r1, c1, i1, w1o, E1T, OFF_DS1, OFF_DD1)
    run(r2, c2, i2, w2o, E2T, OFF_DS2, OFF_DD2)
    pltpu.sync_copy(deg_v, degp.at[wid])


@jax.jit
def _deg_call(r1, c1, i1, r2, c2, i2, ew, zer):
    return pl.kernel(
        _deg_body,
        out_type=[
            jax.ShapeDtypeStruct((E1P,), jnp.float32),
            jax.ShapeDtypeStruct((E2P,), jnp.float32),
            jax.ShapeDtypeStruct((NW, DEG_TOT), jnp.float32),
        ],
        mesh=_mesh,
        compiler_params=_sc_params,
        scratch_types=[
            pltpu.VMEM((DEG_TOT,), jnp.float32),
            pltpu.VMEM((E1T,), jnp.int32),
            pltpu.VMEM((E1T,), jnp.int32),
            pltpu.VMEM((E1T,), jnp.int32),
            pltpu.VMEM((E1T,), jnp.float32),
            pltpu.SemaphoreType.DMA,
            pltpu.SemaphoreType.DMA,
            pltpu.SemaphoreType.DMA,
            pltpu.SemaphoreType.DMA,
        ],
    )(r1, c1, i1, r2, c2, i2, ew, zer)


def _edge_group(rbuf, cbuf, wbuf, ds_v, dd_v, agg_v, xsrc, rowidx, gb, g):
    iota = lax.iota(jnp.int32, L)
    r16 = rbuf[pl.ds(gb, L)]
    c16 = cbuf[pl.ds(gb, L)]
    w16 = wbuf[pl.ds(gb, L)]
    a = plsc.load_gather(ds_v, [r16])
    b = plsc.load_gather(dd_v, [c16])
    norm = a * w16 * b
    xi = rowidx(g, r16, iota)
    cb = c16 * F_HID
    xs = [plsc.load_gather(xsrc, [xi, jnp.full((L,), f, jnp.int32)])
          for f in range(F_HID)]
    for f in range(F_HID):
        plsc.addupdate_scatter(agg_v, [cb + f], xs[f] * norm)


def _agg1_body(row, col, w, dinv, xlin, zer,
               aggp,
               agg_v, ds_v, dd_v, rbuf, cbuf, wbuf,
               rows0, rows1, rows2, rows3, sem0, sem1, sem2, sem3):
    wid = _wid()
    pltpu.sync_copy(zer.at[pl.ds(0, N_DST1 * F_HID)], agg_v)
    pltpu.sync_copy(dinv.at[pl.ds(OFF_DS1, N_SRC0)], ds_v)
    pltpu.sync_copy(dinv.at[pl.ds(OFF_DD1, N_DST1)], dd_v)
    base = wid * E1T
    pltpu.sync_copy(row.at[pl.ds(base, E1T)], rbuf)
    pltpu.sync_copy(col.at[pl.ds(base, E1T)], cbuf)
    pltpu.sync_copy(w.at[pl.ds(base, E1T)], wbuf)
    nsub = E1T // SUB
    bufs = (rows0, rows1, rows2, rows3)
    sems = (sem0, sem1, sem2, sem3)

    def fire(s, b):
        pltpu.async_copy(xlin.at[rbuf.at[pl.ds(s * SUB, SUB)]],
                         bufs[b], sems[b])

    def drain(b):
        pltpu.make_async_copy(xlin.at[pl.ds(0, SUB)], bufs[b], sems[b]).wait()

    def compute(s, b):
        rows_v = bufs[b]

        @plsc.parallel_loop(0, SUB // L, 1)
        def grp(g):
            _edge_group(rbuf, cbuf, wbuf, ds_v, dd_v, agg_v, rows_v,
                        lambda g_, r16, iota: g_ * L + iota, s * SUB + g * L, g)

    for b in range(3):
        fire(b, b)

    def outer(k, carry):
        s = k * 4
        for b in range(4):
            sb = s + b

            @pl.when(sb + 3 < nsub)
            def _():
                fire(sb + 3, (b + 3) % 4)

            drain(b)
            compute(sb, b)
        return carry

    lax.fori_loop(0, nsub // 4, outer, 0)
    pltpu.sync_copy(agg_v, aggp.at[wid])


@jax.jit
def _agg1_call(row, col, w, dinv, xlin, zer):
    return pl.kernel(
        _agg1_body,
        out_type=jax.ShapeDtypeStruct((NW, N_DST1 * F_HID), jnp.float32),
        mesh=_mesh,
        compiler_params=_sc_params,
        scratch_types=[
            pltpu.VMEM((N_DST1 * F_HID,), jnp.float32),
            pltpu.VMEM((N_SRC0,), jnp.float32),
            pltpu.VMEM((N_DST1,), jnp.float32),
            pltpu.VMEM((E1T,), jnp.int32),
            pltpu.VMEM((E1T,), jnp.int32),
            pltpu.VMEM((E1T,), jnp.float32),
            pltpu.VMEM((SUB, F_HID), jnp.float32),
            pltpu.VMEM((SUB, F_HID), jnp.float32),
            pltpu.VMEM((SUB, F_HID), jnp.float32),
            pltpu.VMEM((SUB, F_HID), jnp.float32),
            pltpu.SemaphoreType.DMA,
            pltpu.SemaphoreType.DMA,
            pltpu.SemaphoreType.DMA,
            pltpu.SemaphoreType.DMA,
        ],
    )(row, col, w, dinv, xlin, zer)


def _agg2_body(row, col, w, dinv, h2d, zer,
               aggp,
               agg_v, ds_v, dd_v, rbuf, cbuf, wbuf, h_v):
    wid = _wid()
    pltpu.sync_copy(zer.at[pl.ds(0, N_DST2 * F_HID)], agg_v)
    pltpu.sync_copy(dinv.at[pl.ds(OFF_DS2, N_DST1)], ds_v)
    pltpu.sync_copy(dinv.at[pl.ds(OFF_DD2, N_DST2)], dd_v)
    pltpu.sync_copy(h2d, h_v)
    base = wid * E2T
    pltpu.sync_copy(row.at[pl.ds(base, E2T)], rbuf)
    pltpu.sync_copy(col.at[pl.ds(base, E2T)], cbuf)
    pltpu.sync_copy(w.at[pl.ds(base, E2T)], wbuf)

    @plsc.parallel_loop(0, E2T // L, 1)
    def grp(g):
        _edge_group(rbuf, cbuf, wbuf, ds_v, dd_v, agg_v, h_v,
                    lambda g_, r16, iota: r16, g * L, g)

    pltpu.sync_copy(agg_v, aggp.at[wid])


@jax.jit
def _agg2_call(row, col, w, dinv, h2d, zer):
    return pl.kernel(
        _agg2_body,
        out_type=jax.ShapeDtypeStruct((NW, N_DST2 * F_HID), jnp.float32),
        mesh=_mesh,
        compiler_params=_sc_params,
        scratch_types=[
            pltpu.VMEM((N_DST2 * F_HID,), jnp.float32),
            pltpu.VMEM((N_DST1,), jnp.float32),
            pltpu.VMEM((N_DST2,), jnp.float32),
            pltpu.VMEM((E2T,), jnp.int32),
            pltpu.VMEM((E2T,), jnp.int32),
            pltpu.VMEM((E2T,), jnp.float32),
            pltpu.VMEM((N_DST1, F_HID), jnp.float32),
        ],
    )(row, col, w, dinv, h2d, zer)


def _tc_prep_body(x_ref, w1_ref, degp_ref, xlin_ref, dinv_ref):
    xlin_ref[...] = jnp.dot(x_ref[...], w1_ref[...],
                            preferred_element_type=jnp.float32)
    deg = jnp.sum(degp_ref[...], axis=0)
    dinv_ref[...] = jnp.where(deg > 0.0, lax.rsqrt(deg), 0.0)


@jax.jit
def _tc_prep(x, W1, degp):
    return pl.pallas_call(
        _tc_prep_body,
        out_shape=[
            jax.ShapeDtypeStruct((N_SRC0, F_HID), jnp.float32),
            jax.ShapeDtypeStruct((151, 128), jnp.float32),
        ],
    )(x, W1, degp)


def _tc_relu_body(aggp_ref, b1_ref, h_ref):
    s = jnp.sum(aggp_ref[...], axis=0)
    h_ref[...] = jnp.maximum(s + b1_ref[...], 0.0)


@jax.jit
def _tc_relu(aggp, b1t):
    return pl.pallas_call(
        _tc_relu_body,
        out_shape=jax.ShapeDtypeStruct((N_DST1 * F_HID // 128, 128),
                                       jnp.float32),
    )(aggp, b1t)


def _tc_final_body(aggp_ref, w2_ref, b2_ref, out_ref):
    agg = jnp.sum(aggp_ref[...], axis=0)
    o = jnp.dot(agg, w2_ref[...], preferred_element_type=jnp.float32)
    o = o + b2_ref[...]
    z = o - jnp.max(o, axis=1, keepdims=True)
    out_ref[...] = z - jnp.log(jnp.sum(jnp.exp(z), axis=1, keepdims=True))


@jax.jit
def _tc_final(aggp, W2, b2):
    return pl.pallas_call(
        _tc_final_body,
        out_shape=jax.ShapeDtypeStruct((N_DST2, F_OUT), jnp.float32),
    )(aggp, W2, b2)


def kernel(x, edge_index1, e_id1, edge_index2, e_id2, edge_weight,
           W1, b1, W2, b2):
    i32 = jnp.int32
    loops1 = jnp.arange(N_DST1, dtype=i32)
    loops2 = jnp.arange(N_DST2, dtype=i32)
    pad1 = E1P - E1 - N_DST1
    pad2 = E2P - E2 - N_DST2
    z1 = jnp.zeros((pad1,), i32)
    z2 = jnp.zeros((pad2,), i32)
    r1 = jnp.concatenate([edge_index1[0], loops1, z1])
    c1 = jnp.concatenate([edge_index1[1], loops1, z1])
    i1 = jnp.concatenate([e_id1, jnp.full((N_DST1,), ONE_ID, i32),
                          jnp.full((pad1,), ZERO_ID, i32)])
    r2 = jnp.concatenate([edge_index2[0], loops2, z2])
    c2 = jnp.concatenate([edge_index2[1], loops2, z2])
    i2 = jnp.concatenate([e_id2, jnp.full((N_DST2,), ONE_ID, i32),
                          jnp.full((pad2,), ZERO_ID, i32)])
    ew = jnp.concatenate([
        edge_weight,
        jnp.array([0.0, 1.0], jnp.float32),
        jnp.zeros((EW_PAD - E_TOT - 2,), jnp.float32),
    ])
    zer = jnp.zeros((N_DST1 * F_HID,), jnp.float32)

    w1, w2, degp = _deg_call(r1, c1, i1, r2, c2, i2, ew, zer)
    xlin, dinv2d = _tc_prep(x, W1, degp.reshape(NW, 151, 128))
    dinv = dinv2d.reshape(DEG_TOT)
    aggp1 = _agg1_call(r1, c1, w1, dinv, xlin, zer)
    b1t = jnp.tile(b1, 128 // F_HID).reshape(1, 128)
    h = _tc_relu(aggp1.reshape(NW, N_DST1 * F_HID // 128, 128), b1t)
    aggp2 = _agg2_call(r2, c2, w2, dinv, h.reshape(N_DST1, F_HID), zer)
    return _tc_final(aggp2.reshape(NW, N_DST2, F_HID), W2,
                     b2.reshape(1, F_OUT))


# speedup vs baseline: 24.8866x; 1.1234x over previous
import functools

import jax
import jax.numpy as jnp
from jax import lax
from jax.experimental import pallas as pl
from jax.experimental.pallas import tpu as pltpu
from jax.experimental.pallas import tpu_sc as plsc

N_SRC0, N_DST1, N_DST2 = 10000, 4000, 1000
E1, E2, E_TOT = 320000, 64000, 400000
F_IN, F_HID, F_OUT = 128, 16, 64

NC, NS = 2, 16
NW = NC * NS
L = 16

OFF_DS1 = 0
OFF_DD1 = 10112
OFF_DS2 = 14208
OFF_DD2 = 18304
DEG_TOT = 19328

SUB = 128
E1T = 10240
E2T = 2048
E1P = E1T * NW
E2P = E2T * NW
EW_PAD = E_TOT + 8
ZERO_ID = E_TOT
ONE_ID = E_TOT + 1

_mesh = plsc.VectorSubcoreMesh(core_axis_name="c", subcore_axis_name="s")
_sc_params = pltpu.CompilerParams(needs_layout_passes=False,
                                  use_tc_tiling_on_sc=False)


def _wid():
    return lax.axis_index("s") * NC + lax.axis_index("c")


def _deg_body(r1, c1, i1, r2, c2, i2, ew, zer,
              w1o, w2o, degp,
              deg_v, rbuf, cbuf, ibuf, wful, ew_sh, sem0, sem1, sem2, sem3):
    wid = _wid()
    pltpu.sync_copy(zer.at[pl.ds(0, DEG_TOT)], deg_v)
    sems = (sem0, sem1, sem2, sem3)

    @pl.when(lax.axis_index("s") == 0)
    def _():
        pltpu.sync_copy(ew, ew_sh)

    plsc.subcore_barrier()

    def run(row, col, eid, wout, n_tile, off_r, off_c):
        base = wid * n_tile
        nsub = n_tile // SUB
        pltpu.sync_copy(row.at[pl.ds(base, n_tile)], rbuf.at[pl.ds(0, n_tile)])
        pltpu.sync_copy(col.at[pl.ds(base, n_tile)], cbuf.at[pl.ds(0, n_tile)])
        pltpu.sync_copy(eid.at[pl.ds(base, n_tile)], ibuf.at[pl.ds(0, n_tile)])

        def fire(s, sem):
            pltpu.async_copy(ew_sh.at[ibuf.at[pl.ds(s * SUB, SUB)]],
                             wful.at[pl.ds(s * SUB, SUB)], sem)

        def drain(s, sem):
            pltpu.make_async_copy(ew.at[pl.ds(0, SUB)],
                                  wful.at[pl.ds(s * SUB, SUB)], sem).wait()

        def compute(s):
            @plsc.parallel_loop(0, SUB // L, 1)
            def grp(g):
                gb = s * SUB + g * L
                r16 = rbuf[pl.ds(gb, L)]
                c16 = cbuf[pl.ds(gb, L)]
                if off_r:
                    r16 = r16 + off_r
                c16 = c16 + off_c
                w16 = wful[pl.ds(gb, L)]
                plsc.addupdate_scatter(deg_v, [r16], w16)
                plsc.addupdate_scatter(deg_v, [c16], w16)

        for b in range(3):
            fire(b, sems[b])

        def outer(k, carry):
            s = k * 4
            for b in range(4):
                sb = s + b

                @pl.when(sb + 3 < nsub)
                def _():
                    fire(sb + 3, sems[(b + 3) % 4])

                drain(sb, sems[b])
                compute(sb)
            return carry

        lax.fori_loop(0, nsub // 4, outer, 0)
        pltpu.sync_copy(wful.at[pl.ds(0, n_tile)], wout.at[pl.ds(base, n_tile)])

    run(r1, c1, i1, w1o, E1T, OFF_DS1, OFF_DD1)
    run(r2, c2, i2, w2o, E2T, OFF_DS2, OFF_DD2)
    pltpu.sync_copy(deg_v, degp.at[wid])


@jax.jit
def _deg_call(r1, c1, i1, r2, c2, i2, ew, zer):
    return pl.kernel(
        _deg_body,
        out_type=[
            jax.ShapeDtypeStruct((E1P,), jnp.float32),
            jax.ShapeDtypeStruct((E2P,), jnp.float32),
            jax.ShapeDtypeStruct((NW, DEG_TOT), jnp.float32),
        ],
        mesh=_mesh,
        compiler_params=_sc_params,
        scratch_types=[
            pltpu.VMEM((DEG_TOT,), jnp.float32),
            pltpu.VMEM((E1T,), jnp.int32),
            pltpu.VMEM((E1T,), jnp.int32),
            pltpu.VMEM((E1T,), jnp.int32),
            pltpu.VMEM((E1T,), jnp.float32),
            pltpu.VMEM_SHARED((EW_PAD,), jnp.float32),
            pltpu.SemaphoreType.DMA,
            pltpu.SemaphoreType.DMA,
            pltpu.SemaphoreType.DMA,
            pltpu.SemaphoreType.DMA,
        ],
    )(r1, c1, i1, r2, c2, i2, ew, zer)


def _edge_group(rbuf, cbuf, wbuf, ds_v, dd_v, agg_v, xsrc, rowidx, gb, g):
    iota = lax.iota(jnp.int32, L)
    r16 = rbuf[pl.ds(gb, L)]
    c16 = cbuf[pl.ds(gb, L)]
    w16 = wbuf[pl.ds(gb, L)]
    a = plsc.load_gather(ds_v, [r16])
    b = plsc.load_gather(dd_v, [c16])
    norm = a * w16 * b
    xi = rowidx(g, r16, iota)
    cb = c16 * F_HID
    xs = [plsc.load_gather(xsrc, [xi, jnp.full((L,), f, jnp.int32)])
          for f in range(F_HID)]
    for f in range(F_HID):
        plsc.addupdate_scatter(agg_v, [cb + f], xs[f] * norm)


def _agg1_body(row, col, w, dinv, xlin, zer,
               aggp,
               agg_v, ds_v, dd_v, rbuf, cbuf, wbuf,
               rows0, rows1, rows2, rows3, xlin_sh,
               sem0, sem1, sem2, sem3):
    wid = _wid()

    @pl.when(lax.axis_index("s") == 0)
    def _():
        pltpu.sync_copy(xlin, xlin_sh)

    pltpu.sync_copy(zer.at[pl.ds(0, N_DST1 * F_HID)], agg_v)
    pltpu.sync_copy(dinv.at[pl.ds(OFF_DS1, N_SRC0)], ds_v)
    pltpu.sync_copy(dinv.at[pl.ds(OFF_DD1, N_DST1)], dd_v)
    base = wid * E1T
    pltpu.sync_copy(row.at[pl.ds(base, E1T)], rbuf)
    pltpu.sync_copy(col.at[pl.ds(base, E1T)], cbuf)
    pltpu.sync_copy(w.at[pl.ds(base, E1T)], wbuf)
    nsub = E1T // SUB
    bufs = (rows0, rows1, rows2, rows3)
    sems = (sem0, sem1, sem2, sem3)
    plsc.subcore_barrier()

    def fire(s, b):
        pltpu.async_copy(xlin_sh.at[rbuf.at[pl.ds(s * SUB, SUB)]],
                         bufs[b], sems[b])

    def drain(b):
        pltpu.make_async_copy(xlin.at[pl.ds(0, SUB)], bufs[b], sems[b]).wait()

    def compute(s, b):
        rows_v = bufs[b]

        @plsc.parallel_loop(0, SUB // L, 1)
        def grp(g):
            _edge_group(rbuf, cbuf, wbuf, ds_v, dd_v, agg_v, rows_v,
                        lambda g_, r16, iota: g_ * L + iota, s * SUB + g * L, g)

    for b in range(3):
        fire(b, b)

    def outer(k, carry):
        s = k * 4
        for b in range(4):
            sb = s + b

            @pl.when(sb + 3 < nsub)
            def _():
                fire(sb + 3, (b + 3) % 4)

            drain(b)
            compute(sb, b)
        return carry

    lax.fori_loop(0, nsub // 4, outer, 0)
    pltpu.sync_copy(agg_v, aggp.at[wid])


@jax.jit
def _agg1_call(row, col, w, dinv, xlin, zer):
    return pl.kernel(
        _agg1_body,
        out_type=jax.ShapeDtypeStruct((NW, N_DST1 * F_HID), jnp.float32),
        mesh=_mesh,
        compiler_params=_sc_params,
        scratch_types=[
            pltpu.VMEM((N_DST1 * F_HID,), jnp.float32),
            pltpu.VMEM((N_SRC0,), jnp.float32),
            pltpu.VMEM((N_DST1,), jnp.float32),
            pltpu.VMEM((E1T,), jnp.int32),
            pltpu.VMEM((E1T,), jnp.int32),
            pltpu.VMEM((E1T,), jnp.float32),
            pltpu.VMEM((SUB, F_HID), jnp.float32),
            pltpu.VMEM((SUB, F_HID), jnp.float32),
            pltpu.VMEM((SUB, F_HID), jnp.float32),
            pltpu.VMEM((SUB, F_HID), jnp.float32),
            pltpu.VMEM_SHARED((N_SRC0, F_HID), jnp.float32),
            pltpu.SemaphoreType.DMA,
            pltpu.SemaphoreType.DMA,
            pltpu.SemaphoreType.DMA,
            pltpu.SemaphoreType.DMA,
        ],
    )(row, col, w, dinv, xlin, zer)


def _agg2_body(row, col, w, dinv, h2d, zer,
               aggp,
               agg_v, ds_v, dd_v, rbuf, cbuf, wbuf, h_v):
    wid = _wid()
    pltpu.sync_copy(zer.at[pl.ds(0, N_DST2 * F_HID)], agg_v)
    pltpu.sync_copy(dinv.at[pl.ds(OFF_DS2, N_DST1)], ds_v)
    pltpu.sync_copy(dinv.at[pl.ds(OFF_DD2, N_DST2)], dd_v)
    pltpu.sync_copy(h2d, h_v)
    base = wid * E2T
    pltpu.sync_copy(row.at[pl.ds(base, E2T)], rbuf)
    pltpu.sync_copy(col.at[pl.ds(base, E2T)], cbuf)
    pltpu.sync_copy(w.at[pl.ds(base, E2T)], wbuf)

    @plsc.parallel_loop(0, E2T // L, 1)
    def grp(g):
        _edge_group(rbuf, cbuf, wbuf, ds_v, dd_v, agg_v, h_v,
                    lambda g_, r16, iota: r16, g * L, g)

    pltpu.sync_copy(agg_v, aggp.at[wid])


@jax.jit
def _agg2_call(row, col, w, dinv, h2d, zer):
    return pl.kernel(
        _agg2_body,
        out_type=jax.ShapeDtypeStruct((NW, N_DST2 * F_HID), jnp.float32),
        mesh=_mesh,
        compiler_params=_sc_params,
        scratch_types=[
            pltpu.VMEM((N_DST2 * F_HID,), jnp.float32),
            pltpu.VMEM((N_DST1,), jnp.float32),
            pltpu.VMEM((N_DST2,), jnp.float32),
            pltpu.VMEM((E2T,), jnp.int32),
            pltpu.VMEM((E2T,), jnp.int32),
            pltpu.VMEM((E2T,), jnp.float32),
            pltpu.VMEM((N_DST1, F_HID), jnp.float32),
        ],
    )(row, col, w, dinv, h2d, zer)


def _tc_prep_body(x_ref, w1_ref, degp_ref, xlin_ref, dinv_ref):
    xlin_ref[...] = jnp.dot(x_ref[...], w1_ref[...],
                            preferred_element_type=jnp.float32)
    deg = jnp.sum(degp_ref[...], axis=0)
    dinv_ref[...] = jnp.where(deg > 0.0, lax.rsqrt(deg), 0.0)


@jax.jit
def _tc_prep(x, W1, degp):
    return pl.pallas_call(
        _tc_prep_body,
        out_shape=[
            jax.ShapeDtypeStruct((N_SRC0, F_HID), jnp.float32),
            jax.ShapeDtypeStruct((151, 128), jnp.float32),
        ],
    )(x, W1, degp)


def _tc_relu_body(aggp_ref, b1_ref, h_ref):
    s = jnp.sum(aggp_ref[...], axis=0)
    h_ref[...] = jnp.maximum(s + b1_ref[...], 0.0)


@jax.jit
def _tc_relu(aggp, b1t):
    return pl.pallas_call(
        _tc_relu_body,
        out_shape=jax.ShapeDtypeStruct((N_DST1 * F_HID // 128, 128),
                                       jnp.float32),
    )(aggp, b1t)


def _tc_final_body(aggp_ref, w2_ref, b2_ref, out_ref):
    agg = jnp.sum(aggp_ref[...], axis=0)
    o = jnp.dot(agg, w2_ref[...], preferred_element_type=jnp.float32)
    o = o + b2_ref[...]
    z = o - jnp.max(o, axis=1, keepdims=True)
    out_ref[...] = z - jnp.log(jnp.sum(jnp.exp(z), axis=1, keepdims=True))


@jax.jit
def _tc_final(aggp, W2, b2):
    return pl.pallas_call(
        _tc_final_body,
        out_shape=jax.ShapeDtypeStruct((N_DST2, F_OUT), jnp.float32),
    )(aggp, W2, b2)


def kernel(x, edge_index1, e_id1, edge_index2, e_id2, edge_weight,
           W1, b1, W2, b2):
    i32 = jnp.int32
    loops1 = jnp.arange(N_DST1, dtype=i32)
    loops2 = jnp.arange(N_DST2, dtype=i32)
    pad1 = E1P - E1 - N_DST1
    pad2 = E2P - E2 - N_DST2
    z1 = jnp.zeros((pad1,), i32)
    z2 = jnp.zeros((pad2,), i32)
    r1 = jnp.concatenate([edge_index1[0], loops1, z1])
    c1 = jnp.concatenate([edge_index1[1], loops1, z1])
    i1 = jnp.concatenate([e_id1, jnp.full((N_DST1,), ONE_ID, i32),
                          jnp.full((pad1,), ZERO_ID, i32)])
    r2 = jnp.concatenate([edge_index2[0], loops2, z2])
    c2 = jnp.concatenate([edge_index2[1], loops2, z2])
    i2 = jnp.concatenate([e_id2, jnp.full((N_DST2,), ONE_ID, i32),
                          jnp.full((pad2,), ZERO_ID, i32)])
    ew = jnp.concatenate([
        edge_weight,
        jnp.array([0.0, 1.0], jnp.float32),
        jnp.zeros((EW_PAD - E_TOT - 2,), jnp.float32),
    ])
    zer = jnp.zeros((N_DST1 * F_HID,), jnp.float32)

    w1, w2, degp = _deg_call(r1, c1, i1, r2, c2, i2, ew, zer)
    xlin, dinv2d = _tc_prep(x, W1, degp.reshape(NW, 151, 128))
    dinv = dinv2d.reshape(DEG_TOT)
    aggp1 = _agg1_call(r1, c1, w1, dinv, xlin, zer)
    b1t = jnp.tile(b1, 128 // F_HID).reshape(1, 128)
    h = _tc_relu(aggp1.reshape(NW, N_DST1 * F_HID // 128, 128), b1t)
    aggp2 = _agg2_call(r2, c2, w2, dinv, h.reshape(N_DST1, F_HID), zer)
    return _tc_final(aggp2.reshape(NW, N_DST2, F_HID), W2,
                     b2.reshape(1, F_OUT))


# speedup vs baseline: 28.8552x; 1.1595x over previous
import functools

import jax
import jax.numpy as jnp
from jax import lax
from jax.experimental import pallas as pl
from jax.experimental.pallas import tpu as pltpu
from jax.experimental.pallas import tpu_sc as plsc

N_SRC0, N_DST1, N_DST2 = 10000, 4000, 1000
E1, E2, E_TOT = 320000, 64000, 400000
F_IN, F_HID, F_OUT = 128, 16, 64

NC, NS = 2, 16
NW = NC * NS
L = 16

OFF_DS1 = 0
OFF_DD1 = 10112
OFF_DS2 = 14208
OFF_DD2 = 18304
DEG_TOT = 19328

SUB = 128
E1T = 10240
E2T = 2048
E1P = E1T * NW
E2P = E2T * NW
EW_PAD = E_TOT + 8
ZERO_ID = E_TOT
ONE_ID = E_TOT + 1

_mesh = plsc.VectorSubcoreMesh(core_axis_name="c", subcore_axis_name="s")
_sc_params = pltpu.CompilerParams(needs_layout_passes=False,
                                  use_tc_tiling_on_sc=False)


def _wid():
    return lax.axis_index("s") * NC + lax.axis_index("c")


def _deg_body(r1, c1, i1, r2, c2, i2, ew, zer,
              w1o, w2o, degp,
              deg_v, rbuf, cbuf, ibuf, wful, ew_sh, sem0, sem1, sem2, sem3):
    wid = _wid()
    pltpu.sync_copy(zer.at[pl.ds(0, DEG_TOT)], deg_v)
    sems = (sem0, sem1, sem2, sem3)

    @pl.when(lax.axis_index("s") == 0)
    def _():
        pltpu.sync_copy(ew, ew_sh)

    plsc.subcore_barrier()

    def run(row, col, eid, wout, n_tile, off_r, off_c):
        base = wid * n_tile
        nsub = n_tile // SUB
        pltpu.sync_copy(row.at[pl.ds(base, n_tile)], rbuf.at[pl.ds(0, n_tile)])
        pltpu.sync_copy(col.at[pl.ds(base, n_tile)], cbuf.at[pl.ds(0, n_tile)])
        pltpu.sync_copy(eid.at[pl.ds(base, n_tile)], ibuf.at[pl.ds(0, n_tile)])

        def fire(s, sem):
            pltpu.async_copy(ew_sh.at[ibuf.at[pl.ds(s * SUB, SUB)]],
                             wful.at[pl.ds(s * SUB, SUB)], sem)

        def drain(s, sem):
            pltpu.make_async_copy(ew.at[pl.ds(0, SUB)],
                                  wful.at[pl.ds(s * SUB, SUB)], sem).wait()

        def compute(s):
            @plsc.parallel_loop(0, SUB // L, 1)
            def grp(g):
                gb = s * SUB + g * L
                r16 = rbuf[pl.ds(gb, L)]
                c16 = cbuf[pl.ds(gb, L)]
                if off_r:
                    r16 = r16 + off_r
                c16 = c16 + off_c
                w16 = wful[pl.ds(gb, L)]
                plsc.addupdate_scatter(deg_v, [r16], w16)
                plsc.addupdate_scatter(deg_v, [c16], w16)

        for b in range(3):
            fire(b, sems[b])

        def outer(k, carry):
            s = k * 4
            for b in range(4):
                sb = s + b

                @pl.when(sb + 3 < nsub)
                def _():
                    fire(sb + 3, sems[(b + 3) % 4])

                drain(sb, sems[b])
                compute(sb)
            return carry

        lax.fori_loop(0, nsub // 4, outer, 0)
        pltpu.sync_copy(wful.at[pl.ds(0, n_tile)], wout.at[pl.ds(base, n_tile)])

    run(r1, c1, i1, w1o, E1T, OFF_DS1, OFF_DD1)
    run(r2, c2, i2, w2o, E2T, OFF_DS2, OFF_DD2)
    pltpu.sync_copy(deg_v, degp.at[wid])


@jax.jit
def _deg_call(r1, c1, i1, r2, c2, i2, ew, zer):
    return pl.kernel(
        _deg_body,
        out_type=[
            jax.ShapeDtypeStruct((E1P,), jnp.float32),
            jax.ShapeDtypeStruct((E2P,), jnp.float32),
            jax.ShapeDtypeStruct((NW, DEG_TOT), jnp.float32),
        ],
        mesh=_mesh,
        compiler_params=_sc_params,
        scratch_types=[
            pltpu.VMEM((DEG_TOT,), jnp.float32),
            pltpu.VMEM((E1T,), jnp.int32),
            pltpu.VMEM((E1T,), jnp.int32),
            pltpu.VMEM((E1T,), jnp.int32),
            pltpu.VMEM((E1T,), jnp.float32),
            pltpu.VMEM_SHARED((EW_PAD,), jnp.float32),
            pltpu.SemaphoreType.DMA,
            pltpu.SemaphoreType.DMA,
            pltpu.SemaphoreType.DMA,
            pltpu.SemaphoreType.DMA,
        ],
    )(r1, c1, i1, r2, c2, i2, ew, zer)


def _edge_group(rbuf, cbuf, wbuf, ds_v, dd_v, agg2d, xsrc2d, rowat, gb):
    r16 = rbuf[pl.ds(gb, L)]
    c16 = cbuf[pl.ds(gb, L)]
    w16 = wbuf[pl.ds(gb, L)]
    a = plsc.load_gather(ds_v, [r16])
    b = plsc.load_gather(dd_v, [c16])
    norm = a * w16 * b
    for e in range(L):
        n_s = norm[e]
        c_s = c16[e]
        xrow = xsrc2d[rowat(e, r16), :]
        plsc.addupdate(agg2d.at[c_s], xrow * n_s)


def _agg1_body(row, col, w, dinv, xlin, zer,
               aggp,
               agg_v, ds_v, dd_v, rbuf, cbuf, wbuf,
               rows0, rows1, rows2, rows3, xlin_sh,
               sem0, sem1, sem2, sem3):
    wid = _wid()

    @pl.when(lax.axis_index("s") == 0)
    def _():
        pltpu.sync_copy(xlin, xlin_sh)

    pltpu.sync_copy(zer, agg_v)
    pltpu.sync_copy(dinv.at[pl.ds(OFF_DS1, N_SRC0)], ds_v)
    pltpu.sync_copy(dinv.at[pl.ds(OFF_DD1, N_DST1)], dd_v)
    base = wid * E1T
    pltpu.sync_copy(row.at[pl.ds(base, E1T)], rbuf)
    pltpu.sync_copy(col.at[pl.ds(base, E1T)], cbuf)
    pltpu.sync_copy(w.at[pl.ds(base, E1T)], wbuf)
    nsub = E1T // SUB
    bufs = (rows0, rows1, rows2, rows3)
    sems = (sem0, sem1, sem2, sem3)
    plsc.subcore_barrier()

    def fire(s, b):
        pltpu.async_copy(xlin_sh.at[rbuf.at[pl.ds(s * SUB, SUB)]],
                         bufs[b], sems[b])

    def drain(b):
        pltpu.make_async_copy(xlin.at[pl.ds(0, SUB)], bufs[b], sems[b]).wait()

    def compute(s, b):
        rows_v = bufs[b]

        @plsc.parallel_loop(0, SUB // L, 1)
        def grp(g):
            _edge_group(rbuf, cbuf, wbuf, ds_v, dd_v, agg_v, rows_v,
                        lambda e, r16: g * L + e, s * SUB + g * L)

    for b in range(3):
        fire(b, b)

    def outer(k, carry):
        s = k * 4
        for b in range(4):
            sb = s + b

            @pl.when(sb + 3 < nsub)
            def _():
                fire(sb + 3, (b + 3) % 4)

            drain(b)
            compute(sb, b)
        return carry

    lax.fori_loop(0, nsub // 4, outer, 0)
    pltpu.sync_copy(agg_v, aggp.at[wid])


@jax.jit
def _agg1_call(row, col, w, dinv, xlin, zer):
    return pl.kernel(
        _agg1_body,
        out_type=jax.ShapeDtypeStruct((NW, N_DST1, F_HID), jnp.float32),
        mesh=_mesh,
        compiler_params=_sc_params,
        scratch_types=[
            pltpu.VMEM((N_DST1, F_HID), jnp.float32),
            pltpu.VMEM((N_SRC0,), jnp.float32),
            pltpu.VMEM((N_DST1,), jnp.float32),
            pltpu.VMEM((E1T,), jnp.int32),
            pltpu.VMEM((E1T,), jnp.int32),
            pltpu.VMEM((E1T,), jnp.float32),
            pltpu.VMEM((SUB, F_HID), jnp.float32),
            pltpu.VMEM((SUB, F_HID), jnp.float32),
            pltpu.VMEM((SUB, F_HID), jnp.float32),
            pltpu.VMEM((SUB, F_HID), jnp.float32),
            pltpu.VMEM_SHARED((N_SRC0, F_HID), jnp.float32),
            pltpu.SemaphoreType.DMA,
            pltpu.SemaphoreType.DMA,
            pltpu.SemaphoreType.DMA,
            pltpu.SemaphoreType.DMA,
        ],
    )(row, col, w, dinv, xlin, zer)


def _agg2_body(row, col, w, dinv, h2d, zer,
               aggp,
               agg_v, ds_v, dd_v, rbuf, cbuf, wbuf, h_v):
    wid = _wid()
    pltpu.sync_copy(zer.at[pl.ds(0, N_DST2), :], agg_v)
    pltpu.sync_copy(dinv.at[pl.ds(OFF_DS2, N_DST1)], ds_v)
    pltpu.sync_copy(dinv.at[pl.ds(OFF_DD2, N_DST2)], dd_v)
    pltpu.sync_copy(h2d, h_v)
    base = wid * E2T
    pltpu.sync_copy(row.at[pl.ds(base, E2T)], rbuf)
    pltpu.sync_copy(col.at[pl.ds(base, E2T)], cbuf)
    pltpu.sync_copy(w.at[pl.ds(base, E2T)], wbuf)

    @plsc.parallel_loop(0, E2T // L, 1)
    def grp(g):
        gb = g * L
        _edge_group(rbuf, cbuf, wbuf, ds_v, dd_v, agg_v, h_v,
                    lambda e, r16: r16[e], gb)

    pltpu.sync_copy(agg_v, aggp.at[wid])


@jax.jit
def _agg2_call(row, col, w, dinv, h2d, zer):
    return pl.kernel(
        _agg2_body,
        out_type=jax.ShapeDtypeStruct((NW, N_DST2, F_HID), jnp.float32),
        mesh=_mesh,
        compiler_params=_sc_params,
        scratch_types=[
            pltpu.VMEM((N_DST2, F_HID), jnp.float32),
            pltpu.VMEM((N_DST1,), jnp.float32),
            pltpu.VMEM((N_DST2,), jnp.float32),
            pltpu.VMEM((E2T,), jnp.int32),
            pltpu.VMEM((E2T,), jnp.int32),
            pltpu.VMEM((E2T,), jnp.float32),
            pltpu.VMEM((N_DST1, F_HID), jnp.float32),
        ],
    )(row, col, w, dinv, h2d, zer)


def _tc_prep_body(x_ref, w1_ref, degp_ref, xlin_ref, dinv_ref):
    xlin_ref[...] = jnp.dot(x_ref[...], w1_ref[...],
                            preferred_element_type=jnp.float32)
    deg = jnp.sum(degp_ref[...], axis=0)
    dinv_ref[...] = jnp.where(deg > 0.0, lax.rsqrt(deg), 0.0)


@jax.jit
def _tc_prep(x, W1, degp):
    return pl.pallas_call(
        _tc_prep_body,
        out_shape=[
            jax.ShapeDtypeStruct((N_SRC0, F_HID), jnp.float32),
            jax.ShapeDtypeStruct((151, 128), jnp.float32),
        ],
    )(x, W1, degp)


def _tc_relu_body(aggp_ref, b1_ref, h_ref):
    s = jnp.sum(aggp_ref[...], axis=0)
    h_ref[...] = jnp.maximum(s + b1_ref[...], 0.0)


@jax.jit
def _tc_relu(aggp, b1t):
    return pl.pallas_call(
        _tc_relu_body,
        out_shape=jax.ShapeDtypeStruct((N_DST1 * F_HID // 128, 128),
                                       jnp.float32),
    )(aggp, b1t)


def _tc_final_body(aggp_ref, w2_ref, b2_ref, out_ref):
    agg = jnp.sum(aggp_ref[...], axis=0)
    o = jnp.dot(agg, w2_ref[...], preferred_element_type=jnp.float32)
    o = o + b2_ref[...]
    z = o - jnp.max(o, axis=1, keepdims=True)
    out_ref[...] = z - jnp.log(jnp.sum(jnp.exp(z), axis=1, keepdims=True))


@jax.jit
def _tc_final(aggp, W2, b2):
    return pl.pallas_call(
        _tc_final_body,
        out_shape=jax.ShapeDtypeStruct((N_DST2, F_OUT), jnp.float32),
    )(aggp, W2, b2)


def kernel(x, edge_index1, e_id1, edge_index2, e_id2, edge_weight,
           W1, b1, W2, b2):
    i32 = jnp.int32
    loops1 = jnp.arange(N_DST1, dtype=i32)
    loops2 = jnp.arange(N_DST2, dtype=i32)
    pad1 = E1P - E1 - N_DST1
    pad2 = E2P - E2 - N_DST2
    z1 = jnp.zeros((pad1,), i32)
    z2 = jnp.zeros((pad2,), i32)
    r1 = jnp.concatenate([edge_index1[0], loops1, z1])
    c1 = jnp.concatenate([edge_index1[1], loops1, z1])
    i1 = jnp.concatenate([e_id1, jnp.full((N_DST1,), ONE_ID, i32),
                          jnp.full((pad1,), ZERO_ID, i32)])
    r2 = jnp.concatenate([edge_index2[0], loops2, z2])
    c2 = jnp.concatenate([edge_index2[1], loops2, z2])
    i2 = jnp.concatenate([e_id2, jnp.full((N_DST2,), ONE_ID, i32),
                          jnp.full((pad2,), ZERO_ID, i32)])
    ew = jnp.concatenate([
        edge_weight,
        jnp.array([0.0, 1.0], jnp.float32),
        jnp.zeros((EW_PAD - E_TOT - 2,), jnp.float32),
    ])
    zer = jnp.zeros((N_DST1 * F_HID,), jnp.float32)
    zer2d = jnp.zeros((N_DST1, F_HID), jnp.float32)

    w1, w2, degp = _deg_call(r1, c1, i1, r2, c2, i2, ew, zer)
    xlin, dinv2d = _tc_prep(x, W1, degp.reshape(NW, 151, 128))
    dinv = dinv2d.reshape(DEG_TOT)
    aggp1 = _agg1_call(r1, c1, w1, dinv, xlin, zer2d)
    b1t = jnp.tile(b1, 128 // F_HID).reshape(1, 128)
    h = _tc_relu(aggp1.reshape(NW, N_DST1 * F_HID // 128, 128), b1t)
    aggp2 = _agg2_call(r2, c2, w2, dinv, h.reshape(N_DST1, F_HID), zer2d)
    return _tc_final(aggp2, W2, b2.reshape(1, F_OUT))


# speedup vs baseline: 29.0189x; 1.0057x over previous
import functools

import jax
import jax.numpy as jnp
from jax import lax
from jax.experimental import pallas as pl
from jax.experimental.pallas import tpu as pltpu
from jax.experimental.pallas import tpu_sc as plsc

N_SRC0, N_DST1, N_DST2 = 10000, 4000, 1000
E1, E2, E_TOT = 320000, 64000, 400000
F_IN, F_HID, F_OUT = 128, 16, 64

NC, NS = 2, 16
NW = NC * NS
L = 16

OFF_DS1 = 0
OFF_DD1 = 10112
OFF_DS2 = 14208
OFF_DD2 = 18304
DEG_ROWS = 256
DEG_TOT = DEG_ROWS * 128

ND1P = 4096
ND2P = 1024

SUB = 128
E1T = 10240
E2T = 2048
E1P = E1T * NW
E2P = E2T * NW
EW_PAD = E_TOT + 8
ZERO_ID = E_TOT
ONE_ID = E_TOT + 1

_mesh = plsc.VectorSubcoreMesh(core_axis_name="c", subcore_axis_name="s")
_sc_params = pltpu.CompilerParams(needs_layout_passes=False,
                                  use_tc_tiling_on_sc=False)


def _wid():
    return lax.axis_index("s") * NC + lax.axis_index("c")


def _fill_rowidx(iidx, n_chunks):
    iota = lax.iota(jnp.int32, L)
    for k in range(n_chunks):
        for j in range(128 // L):
            iidx[k, pl.ds(j * L, L)] = iota + (k * 128 + j * L)


def _add_rows(src_v, dst_sh, iidx, n_rows):
    for k in range(n_rows // 128):
        pltpu.sync_copy(src_v.at[pl.ds(k * 128, 128), :],
                        dst_sh.at[iidx.at[k]], add=True)


def _deg_body(r1, c1, i1, r2, c2, i2, ew, zerd,
              w1o, w2o, degp,
              deg_v, rbuf, cbuf, ibuf, wful, iidx, ew_sh, deg_sh,
              sem0, sem1, sem2, sem3):
    wid = _wid()
    sid = lax.axis_index("s")
    cid = lax.axis_index("c")
    sems = (sem0, sem1, sem2, sem3)
    pltpu.sync_copy(zerd, deg_v)
    _fill_rowidx(iidx, DEG_ROWS // 128)

    @pl.when(sid == 0)
    def _():
        pltpu.sync_copy(ew, ew_sh)
        pltpu.sync_copy(zerd, deg_sh)

    plsc.subcore_barrier()

    def run(row, col, eid, wout, n_tile, off_r, off_c):
        base = wid * n_tile
        nsub = n_tile // SUB
        pltpu.sync_copy(row.at[pl.ds(base, n_tile)], rbuf.at[pl.ds(0, n_tile)])
        pltpu.sync_copy(col.at[pl.ds(base, n_tile)], cbuf.at[pl.ds(0, n_tile)])
        pltpu.sync_copy(eid.at[pl.ds(base, n_tile)], ibuf.at[pl.ds(0, n_tile)])

        def fire(s, sem):
            pltpu.async_copy(ew_sh.at[ibuf.at[pl.ds(s * SUB, SUB)]],
                             wful.at[pl.ds(s * SUB, SUB)], sem)

        def drain(s, sem):
            pltpu.make_async_copy(ew.at[pl.ds(0, SUB)],
                                  wful.at[pl.ds(s * SUB, SUB)], sem).wait()

        def compute(s):
            @plsc.parallel_loop(0, SUB // L, 1)
            def grp(g):
                gb = s * SUB + g * L
                r16 = rbuf[pl.ds(gb, L)]
                c16 = cbuf[pl.ds(gb, L)]
                if off_r:
                    r16 = r16 + off_r
                c16 = c16 + off_c
                w16 = wful[pl.ds(gb, L)]
                plsc.addupdate_scatter(
                    deg_v, [r16 >> 7, r16 & 127], w16)
                plsc.addupdate_scatter(
                    deg_v, [c16 >> 7, c16 & 127], w16)

        for b in range(3):
            fire(b, sems[b])

        def outer(k, carry):
            s = k * 4
            for b in range(4):
                sb = s + b

                @pl.when(sb + 3 < nsub)
                def _():
                    fire(sb + 3, sems[(b + 3) % 4])

                drain(sb, sems[b])
                compute(sb)
            return carry

        lax.fori_loop(0, nsub // 4, outer, 0)
        pltpu.sync_copy(wful.at[pl.ds(0, n_tile)], wout.at[pl.ds(base, n_tile)])

    run(r1, c1, i1, w1o, E1T, OFF_DS1, OFF_DD1)
    run(r2, c2, i2, w2o, E2T, OFF_DS2, OFF_DD2)
    _add_rows(deg_v, deg_sh, iidx, DEG_ROWS)
    plsc.subcore_barrier()

    @pl.when(sid == 0)
    def _():
        pltpu.sync_copy(deg_sh, degp.at[cid])


@jax.jit
def _deg_call(r1, c1, i1, r2, c2, i2, ew, zerd):
    return pl.kernel(
        _deg_body,
        out_type=[
            jax.ShapeDtypeStruct((E1P,), jnp.float32),
            jax.ShapeDtypeStruct((E2P,), jnp.float32),
            jax.ShapeDtypeStruct((NC, DEG_ROWS, 128), jnp.float32),
        ],
        mesh=_mesh,
        compiler_params=_sc_params,
        scratch_types=[
            pltpu.VMEM((DEG_ROWS, 128), jnp.float32),
            pltpu.VMEM((E1T,), jnp.int32),
            pltpu.VMEM((E1T,), jnp.int32),
            pltpu.VMEM((E1T,), jnp.int32),
            pltpu.VMEM((E1T,), jnp.float32),
            pltpu.VMEM((DEG_ROWS // 128, 128), jnp.int32),
            pltpu.VMEM_SHARED((EW_PAD,), jnp.float32),
            pltpu.VMEM_SHARED((DEG_ROWS, 128), jnp.float32),
            pltpu.SemaphoreType.DMA,
            pltpu.SemaphoreType.DMA,
            pltpu.SemaphoreType.DMA,
            pltpu.SemaphoreType.DMA,
        ],
    )(r1, c1, i1, r2, c2, i2, ew, zerd)


def _edge_group(rbuf, cbuf, wbuf, ds_v, dd_v, agg2d, xsrc2d, rowat, gb):
    r16 = rbuf[pl.ds(gb, L)]
    c16 = cbuf[pl.ds(gb, L)]
    w16 = wbuf[pl.ds(gb, L)]
    a = plsc.load_gather(ds_v, [r16])
    b = plsc.load_gather(dd_v, [c16])
    norm = a * w16 * b
    for e in range(L):
        n_s = norm[e]
        c_s = c16[e]
        xrow = xsrc2d[rowat(e, r16), :]
        plsc.addupdate(agg2d.at[c_s], xrow * n_s)


def _zero2d(zsmall, dst, n_rows):
    for k in range(n_rows // 128):
        pltpu.sync_copy(zsmall, dst.at[pl.ds(k * 128, 128), :])


def _agg1_body(row, col, w, dinv, xlin, zsmall,
               aggp,
               agg_v, ds_v, dd_v, rbuf, cbuf, wbuf,
               rows0, rows1, iidx, agg_sh,
               sem0, sem1):
    wid = _wid()
    sid = lax.axis_index("s")
    cid = lax.axis_index("c")

    @pl.when(sid == 0)
    def _():
        _zero2d(zsmall, agg_sh, ND1P)

    _zero2d(zsmall, agg_v, ND1P)
    pltpu.sync_copy(dinv.at[pl.ds(OFF_DS1, N_SRC0)], ds_v)
    pltpu.sync_copy(dinv.at[pl.ds(OFF_DD1, N_DST1)], dd_v)
    base = wid * E1T
    pltpu.sync_copy(row.at[pl.ds(base, E1T)], rbuf)
    pltpu.sync_copy(col.at[pl.ds(base, E1T)], cbuf)
    pltpu.sync_copy(w.at[pl.ds(base, E1T)], wbuf)
    _fill_rowidx(iidx, ND1P // 128)
    nsub = E1T // SUB
    bufs = (rows0, rows1)
    sems = (sem0, sem1)
    plsc.subcore_barrier()

    def fire(s, b):
        pltpu.async_copy(xlin.at[rbuf.at[pl.ds(s * SUB, SUB)]],
                         bufs[b], sems[b])

    def drain(b):
        pltpu.make_async_copy(xlin.at[pl.ds(0, SUB)], bufs[b], sems[b]).wait()

    def compute(s, b):
        rows_v = bufs[b]

        @plsc.parallel_loop(0, SUB // L, 1)
        def grp(g):
            _edge_group(rbuf, cbuf, wbuf, ds_v, dd_v, agg_v, rows_v,
                        lambda e, r16: g * L + e, s * SUB + g * L)

    fire(0, 0)

    def outer(k, carry):
        s = k * 2
        fire(s + 1, 1)
        drain(0)
        compute(s, 0)

        @pl.when(s + 2 < nsub)
        def _():
            fire(s + 2, 0)

        drain(1)
        compute(s + 1, 1)
        return carry

    lax.fori_loop(0, nsub // 2, outer, 0)
    _add_rows(agg_v, agg_sh, iidx, ND1P)
    plsc.subcore_barrier()

    @pl.when(sid == 0)
    def _():
        pltpu.sync_copy(agg_sh, aggp.at[cid])


@jax.jit
def _agg1_call(row, col, w, dinv, xlin, zsmall):
    return pl.kernel(
        _agg1_body,
        out_type=jax.ShapeDtypeStruct((NC, ND1P, F_HID), jnp.float32),
        mesh=_mesh,
        compiler_params=_sc_params,
        scratch_types=[
            pltpu.VMEM((ND1P, F_HID), jnp.float32),
            pltpu.VMEM((N_SRC0,), jnp.float32),
            pltpu.VMEM((N_DST1,), jnp.float32),
            pltpu.VMEM((E1T,), jnp.int32),
            pltpu.VMEM((E1T,), jnp.int32),
            pltpu.VMEM((E1T,), jnp.float32),
            pltpu.VMEM((SUB, F_HID), jnp.float32),
            pltpu.VMEM((SUB, F_HID), jnp.float32),
            pltpu.VMEM((ND1P // 128, 128), jnp.int32),
            pltpu.VMEM_SHARED((ND1P, F_HID), jnp.float32),
            pltpu.SemaphoreType.DMA,
            pltpu.SemaphoreType.DMA,
        ],
    )(row, col, w, dinv, xlin, zsmall)


def _agg2_body(row, col, w, dinv, aggp1, b1, zsmall,
               aggp,
               agg_v, ds_v, dd_v, b1_v, t0, t1, rbuf, cbuf, wbuf, h_v,
               iidx, h_sh, agg_sh):
    wid = _wid()
    sid = lax.axis_index("s")
    cid = lax.axis_index("c")

    @pl.when(sid == 0)
    def _():
        _zero2d(zsmall, agg_sh, ND2P)

    r0 = sid * (ND1P // NS)
    pltpu.sync_copy(aggp1.at[0].at[pl.ds(r0, ND1P // NS), :], t0)
    pltpu.sync_copy(aggp1.at[1].at[pl.ds(r0, ND1P // NS), :], t1)
    pltpu.sync_copy(b1, b1_v)
    b1vec = b1_v[...]

    @plsc.parallel_loop(0, ND1P // NS, 1)
    def hrow(r):
        t0[r, :] = jnp.maximum(t0[r, :] + t1[r, :] + b1vec, 0.0)

    pltpu.sync_copy(t0, h_sh.at[pl.ds(r0, ND1P // NS), :])

    _zero2d(zsmall, agg_v, ND2P)
    pltpu.sync_copy(dinv.at[pl.ds(OFF_DS2, N_DST1)], ds_v)
    pltpu.sync_copy(dinv.at[pl.ds(OFF_DD2, N_DST2)], dd_v)
    base = wid * E2T
    pltpu.sync_copy(row.at[pl.ds(base, E2T)], rbuf)
    pltpu.sync_copy(col.at[pl.ds(base, E2T)], cbuf)
    pltpu.sync_copy(w.at[pl.ds(base, E2T)], wbuf)
    _fill_rowidx(iidx, ND2P // 128)
    plsc.subcore_barrier()

    pltpu.sync_copy(h_sh, h_v)

    @plsc.parallel_loop(0, E2T // L, 1)
    def grp(g):
        gb = g * L
        _edge_group(rbuf, cbuf, wbuf, ds_v, dd_v, agg_v, h_v,
                    lambda e, r16: r16[e], gb)

    _add_rows(agg_v, agg_sh, iidx, ND2P)
    plsc.subcore_barrier()

    @pl.when(sid == 0)
    def _():
        pltpu.sync_copy(agg_sh, aggp.at[cid])


@jax.jit
def _agg2_call(row, col, w, dinv, aggp1, b1, zsmall):
    return pl.kernel(
        _agg2_body,
        out_type=jax.ShapeDtypeStruct((NC, ND2P, F_HID), jnp.float32),
        mesh=_mesh,
        compiler_params=_sc_params,
        scratch_types=[
            pltpu.VMEM((ND2P, F_HID), jnp.float32),
            pltpu.VMEM((N_DST1,), jnp.float32),
            pltpu.VMEM((N_DST2,), jnp.float32),
            pltpu.VMEM((F_HID,), jnp.float32),
            pltpu.VMEM((ND1P // NS, F_HID), jnp.float32),
            pltpu.VMEM((ND1P // NS, F_HID), jnp.float32),
            pltpu.VMEM((E2T,), jnp.int32),
            pltpu.VMEM((E2T,), jnp.int32),
            pltpu.VMEM((E2T,), jnp.float32),
            pltpu.VMEM((ND1P, F_HID), jnp.float32),
            pltpu.VMEM((ND2P // 128, 128), jnp.int32),
            pltpu.VMEM_SHARED((ND1P, F_HID), jnp.float32),
            pltpu.VMEM_SHARED((ND2P, F_HID), jnp.float32),
        ],
    )(row, col, w, dinv, aggp1, b1, zsmall)


def _tc_prep_body(x_ref, w1_ref, degp_ref, xlin_ref, dinv_ref):
    xlin_ref[...] = jnp.dot(x_ref[...], w1_ref[...],
                            preferred_element_type=jnp.float32)
    deg = degp_ref[0] + degp_ref[1]
    dinv_ref[...] = jnp.where(deg > 0.0, lax.rsqrt(deg), 0.0)


@jax.jit
def _tc_prep(x, W1, degp):
    return pl.pallas_call(
        _tc_prep_body,
        out_shape=[
            jax.ShapeDtypeStruct((N_SRC0, F_HID), jnp.float32),
            jax.ShapeDtypeStruct((DEG_ROWS, 128), jnp.float32),
        ],
    )(x, W1, degp)


def _tc_final_body(aggp_ref, w2_ref, b2_ref, out_ref):
    agg = (aggp_ref[0] + aggp_ref[1])[:N_DST2]
    o = jnp.dot(agg, w2_ref[...], preferred_element_type=jnp.float32)
    o = o + b2_ref[...]
    z = o - jnp.max(o, axis=1, keepdims=True)
    out_ref[...] = z - jnp.log(jnp.sum(jnp.exp(z), axis=1, keepdims=True))


@jax.jit
def _tc_final(aggp, W2, b2):
    return pl.pallas_call(
        _tc_final_body,
        out_shape=jax.ShapeDtypeStruct((N_DST2, F_OUT), jnp.float32),
    )(aggp, W2, b2)


def kernel(x, edge_index1, e_id1, edge_index2, e_id2, edge_weight,
           W1, b1, W2, b2):
    i32 = jnp.int32
    loops1 = jnp.arange(N_DST1, dtype=i32)
    loops2 = jnp.arange(N_DST2, dtype=i32)
    pad1 = E1P - E1 - N_DST1
    pad2 = E2P - E2 - N_DST2
    z1 = jnp.zeros((pad1,), i32)
    z2 = jnp.zeros((pad2,), i32)
    r1 = jnp.concatenate([edge_index1[0], loops1, z1])
    c1 = jnp.concatenate([edge_index1[1], loops1, z1])
    i1 = jnp.concatenate([e_id1, jnp.full((N_DST1,), ONE_ID, i32),
                          jnp.full((pad1,), ZERO_ID, i32)])
    r2 = jnp.concatenate([edge_index2[0], loops2, z2])
    c2 = jnp.concatenate([edge_index2[1], loops2, z2])
    i2 = jnp.concatenate([e_id2, jnp.full((N_DST2,), ONE_ID, i32),
                          jnp.full((pad2,), ZERO_ID, i32)])
    ew = jnp.concatenate([
        edge_weight,
        jnp.array([0.0, 1.0], jnp.float32),
        jnp.zeros((EW_PAD - E_TOT - 2,), jnp.float32),
    ])
    zerd = jnp.zeros((DEG_ROWS, 128), jnp.float32)
    zsmall = jnp.zeros((128, F_HID), jnp.float32)

    w1, w2, degp = _deg_call(r1, c1, i1, r2, c2, i2, ew, zerd)
    xlin, dinv2d = _tc_prep(x, W1, degp)
    dinv = dinv2d.reshape(DEG_TOT)
    aggp1 = _agg1_call(r1, c1, w1, dinv, xlin, zsmall)
    aggp2 = _agg2_call(r2, c2, w2, dinv, aggp1, b1, zsmall)
    return _tc_final(aggp2, W2, b2.reshape(1, F_OUT))


# speedup vs baseline: 31.4295x; 1.0831x over previous
import functools

import jax
import jax.numpy as jnp
from jax import lax
from jax.experimental import pallas as pl
from jax.experimental.pallas import tpu as pltpu
from jax.experimental.pallas import tpu_sc as plsc

N_SRC0, N_DST1, N_DST2 = 10000, 4000, 1000
E1, E2, E_TOT = 320000, 64000, 400000
F_IN, F_HID, F_OUT = 128, 16, 64

NC, NS = 2, 16
NW = NC * NS
L = 16

OFF_DS1 = 0
OFF_DD1 = 10112
OFF_DS2 = 14208
OFF_DD2 = 18304
DEG_ROWS = 256
DEG_TOT = DEG_ROWS * 128

ND1P = 4096
ND2P = 1024

SUB = 128
E1T = 10240
E2T = 2048
E1P = E1T * NW
E2P = E2T * NW
EW_PAD = E_TOT + 8
ZERO_ID = E_TOT
ONE_ID = E_TOT + 1

_mesh = plsc.VectorSubcoreMesh(core_axis_name="c", subcore_axis_name="s")
_sc_params = pltpu.CompilerParams(needs_layout_passes=False,
                                  use_tc_tiling_on_sc=False)


def _wid():
    return lax.axis_index("s") * NC + lax.axis_index("c")


def _add_rows(src_v, dst_sh, iidx, n_rows):
    iota = lax.iota(jnp.int32, L)
    for k in range(n_rows // 128):
        for j in range(128 // L):
            iidx[0, pl.ds(j * L, L)] = iota + (k * 128 + j * L)
        pltpu.sync_copy(src_v.at[pl.ds(k * 128, 128), :],
                        dst_sh.at[iidx.at[0]], add=True)


def _deg_body(r1, c1, i1, r2, c2, i2, ew, zerd,
              w1o, w2o, degp,
              deg_v, rbuf, cbuf, ibuf, wful, iidx, ew_sh, deg_sh,
              sem0, sem1, sem2, sem3):
    wid = _wid()
    sid = lax.axis_index("s")
    cid = lax.axis_index("c")
    sems = (sem0, sem1, sem2, sem3)
    pltpu.sync_copy(zerd, deg_v)

    @pl.when(sid == 0)
    def _():
        pltpu.sync_copy(ew, ew_sh)
        pltpu.sync_copy(zerd, deg_sh)

    plsc.subcore_barrier()

    def run(row, col, eid, wout, n_tile, off_r, off_c):
        base = wid * n_tile
        nsub = n_tile // SUB
        pltpu.sync_copy(row.at[pl.ds(base, n_tile)], rbuf.at[pl.ds(0, n_tile)])
        pltpu.sync_copy(col.at[pl.ds(base, n_tile)], cbuf.at[pl.ds(0, n_tile)])
        pltpu.sync_copy(eid.at[pl.ds(base, n_tile)], ibuf.at[pl.ds(0, n_tile)])

        def fire(s, sem):
            pltpu.async_copy(ew_sh.at[ibuf.at[pl.ds(s * SUB, SUB)]],
                             wful.at[pl.ds(s * SUB, SUB)], sem)

        def drain(s, sem):
            pltpu.make_async_copy(ew.at[pl.ds(0, SUB)],
                                  wful.at[pl.ds(s * SUB, SUB)], sem).wait()

        def compute(s):
            @plsc.parallel_loop(0, SUB // L, 1)
            def grp(g):
                gb = s * SUB + g * L
                r16 = rbuf[pl.ds(gb, L)]
                c16 = cbuf[pl.ds(gb, L)]
                if off_r:
                    r16 = r16 + off_r
                c16 = c16 + off_c
                w16 = wful[pl.ds(gb, L)]
                plsc.addupdate_scatter(
                    deg_v, [r16 >> 7, r16 & 127], w16)
                plsc.addupdate_scatter(
                    deg_v, [c16 >> 7, c16 & 127], w16)

        for b in range(3):
            fire(b, sems[b])

        def outer(k, carry):
            s = k * 4
            for b in range(4):
                sb = s + b

                @pl.when(sb + 3 < nsub)
                def _():
                    fire(sb + 3, sems[(b + 3) % 4])

                drain(sb, sems[b])
                compute(sb)
            return carry

        lax.fori_loop(0, nsub // 4, outer, 0)
        pltpu.sync_copy(wful.at[pl.ds(0, n_tile)], wout.at[pl.ds(base, n_tile)])

    run(r1, c1, i1, w1o, E1T, OFF_DS1, OFF_DD1)
    run(r2, c2, i2, w2o, E2T, OFF_DS2, OFF_DD2)
    _add_rows(deg_v, deg_sh, iidx, DEG_ROWS)
    plsc.subcore_barrier()

    @pl.when(sid == 0)
    def _():
        pltpu.sync_copy(deg_sh, degp.at[cid])


@jax.jit
def _deg_call(r1, c1, i1, r2, c2, i2, ew, zerd):
    return pl.kernel(
        _deg_body,
        out_type=[
            jax.ShapeDtypeStruct((E1P,), jnp.float32),
            jax.ShapeDtypeStruct((E2P,), jnp.float32),
            jax.ShapeDtypeStruct((NC, DEG_ROWS, 128), jnp.float32),
        ],
        mesh=_mesh,
        compiler_params=_sc_params,
        scratch_types=[
            pltpu.VMEM((DEG_ROWS, 128), jnp.float32),
            pltpu.VMEM((E1T,), jnp.int32),
            pltpu.VMEM((E1T,), jnp.int32),
            pltpu.VMEM((E1T,), jnp.int32),
            pltpu.VMEM((E1T,), jnp.float32),
            pltpu.VMEM((1, 128), jnp.int32),
            pltpu.VMEM_SHARED((EW_PAD,), jnp.float32),
            pltpu.VMEM_SHARED((DEG_ROWS, 128), jnp.float32),
            pltpu.SemaphoreType.DMA,
            pltpu.SemaphoreType.DMA,
            pltpu.SemaphoreType.DMA,
            pltpu.SemaphoreType.DMA,
        ],
    )(r1, c1, i1, r2, c2, i2, ew, zerd)


def _edge_group(r16, c16, w16, ds_v, dd_v, agg2d, xsrc2d, rowat):
    a = plsc.load_gather(ds_v, [r16])
    b = plsc.load_gather(dd_v, [c16])
    norm = a * w16 * b
    for e in range(L):
        n_s = norm[e]
        c_s = c16[e]
        xrow = xsrc2d[rowat(e, r16), :]
        plsc.addupdate(agg2d.at[c_s], xrow * n_s)


def _zero2d(zsmall, dst, n_rows):
    for k in range(n_rows // 128):
        pltpu.sync_copy(zsmall, dst.at[pl.ds(k * 128, 128), :])


def _agg1_body(rc, w, dinv, xlin, zsmall,
               aggp,
               agg_v, ds_v, dd_v, rcbuf, rbuf, wbuf,
               rows0, rows1, iidx, xlin_sh, agg_sh,
               sem0, sem1):
    wid = _wid()
    sid = lax.axis_index("s")
    cid = lax.axis_index("c")

    @pl.when(sid == 0)
    def _():
        pltpu.sync_copy(xlin, xlin_sh)
        _zero2d(zsmall, agg_sh, ND1P)

    _zero2d(zsmall, agg_v, ND1P)
    pltpu.sync_copy(dinv.at[pl.ds(OFF_DS1, N_SRC0)], ds_v)
    pltpu.sync_copy(dinv.at[pl.ds(OFF_DD1, N_DST1)], dd_v)
    base = wid * E1T
    pltpu.sync_copy(rc.at[pl.ds(base, E1T)], rcbuf)
    pltpu.sync_copy(w.at[pl.ds(base, E1T)], wbuf)

    @plsc.parallel_loop(0, E1T // L, 1)
    def unpack(u):
        rbuf[pl.ds(u * L, L)] = rcbuf[pl.ds(u * L, L)] >> 14
    nsub = E1T // SUB
    bufs = (rows0, rows1)
    sems = (sem0, sem1)
    plsc.subcore_barrier()

    def fire(s, b):
        pltpu.async_copy(xlin_sh.at[rbuf.at[pl.ds(s * SUB, SUB)]],
                         bufs[b], sems[b])

    def drain(b):
        pltpu.make_async_copy(xlin.at[pl.ds(0, SUB)], bufs[b], sems[b]).wait()

    def compute(s, b):
        rows_v = bufs[b]

        @plsc.parallel_loop(0, SUB // L, 1)
        def grp(g):
            gb = s * SUB + g * L
            rc16 = rcbuf[pl.ds(gb, L)]
            _edge_group(rc16 >> 14, rc16 & 16383, wbuf[pl.ds(gb, L)],
                        ds_v, dd_v, agg_v, rows_v,
                        lambda e, r16: g * L + e)

    fire(0, 0)

    def outer(k, carry):
        s = k * 2
        fire(s + 1, 1)
        drain(0)
        compute(s, 0)

        @pl.when(s + 2 < nsub)
        def _():
            fire(s + 2, 0)

        drain(1)
        compute(s + 1, 1)
        return carry

    lax.fori_loop(0, nsub // 2, outer, 0)
    _add_rows(agg_v, agg_sh, iidx, ND1P)
    plsc.subcore_barrier()

    @pl.when(sid == 0)
    def _():
        pltpu.sync_copy(agg_sh, aggp.at[cid])


@jax.jit
def _agg1_call(rc, w, dinv, xlin, zsmall):
    return pl.kernel(
        _agg1_body,
        out_type=jax.ShapeDtypeStruct((NC, ND1P, F_HID), jnp.float32),
        mesh=_mesh,
        compiler_params=_sc_params,
        scratch_types=[
            pltpu.VMEM((ND1P, F_HID), jnp.float32),
            pltpu.VMEM((N_SRC0,), jnp.float32),
            pltpu.VMEM((N_DST1,), jnp.float32),
            pltpu.VMEM((E1T,), jnp.int32),
            pltpu.VMEM((E1T,), jnp.int32),
            pltpu.VMEM((E1T,), jnp.float32),
            pltpu.VMEM((SUB, F_HID), jnp.float32),
            pltpu.VMEM((SUB, F_HID), jnp.float32),
            pltpu.VMEM((1, 128), jnp.int32),
            pltpu.VMEM_SHARED((N_SRC0, F_HID), jnp.float32),
            pltpu.VMEM_SHARED((ND1P, F_HID), jnp.float32),
            pltpu.SemaphoreType.DMA,
            pltpu.SemaphoreType.DMA,
        ],
    )(rc, w, dinv, xlin, zsmall)


def _agg2_body(row, col, w, dinv, aggp1, b1, zsmall,
               aggp,
               agg_v, ds_v, dd_v, b1_v, t0, t1, rbuf, cbuf, wbuf, h_v,
               iidx, h_sh, agg_sh):
    wid = _wid()
    sid = lax.axis_index("s")
    cid = lax.axis_index("c")

    @pl.when(sid == 0)
    def _():
        _zero2d(zsmall, agg_sh, ND2P)

    r0 = sid * (ND1P // NS)
    pltpu.sync_copy(aggp1.at[0].at[pl.ds(r0, ND1P // NS), :], t0)
    pltpu.sync_copy(aggp1.at[1].at[pl.ds(r0, ND1P // NS), :], t1)
    pltpu.sync_copy(b1, b1_v)
    b1vec = b1_v[...]

    @plsc.parallel_loop(0, ND1P // NS, 1)
    def hrow(r):
        t0[r, :] = jnp.maximum(t0[r, :] + t1[r, :] + b1vec, 0.0)

    pltpu.sync_copy(t0, h_sh.at[pl.ds(r0, ND1P // NS), :])

    _zero2d(zsmall, agg_v, ND2P)
    pltpu.sync_copy(dinv.at[pl.ds(OFF_DS2, N_DST1)], ds_v)
    pltpu.sync_copy(dinv.at[pl.ds(OFF_DD2, N_DST2)], dd_v)
    base = wid * E2T
    pltpu.sync_copy(row.at[pl.ds(base, E2T)], rbuf)
    pltpu.sync_copy(col.at[pl.ds(base, E2T)], cbuf)
    pltpu.sync_copy(w.at[pl.ds(base, E2T)], wbuf)
    plsc.subcore_barrier()

    pltpu.sync_copy(h_sh, h_v)

    @plsc.parallel_loop(0, E2T // L, 1)
    def grp(g):
        gb = g * L
        _edge_group(rbuf[pl.ds(gb, L)], cbuf[pl.ds(gb, L)],
                    wbuf[pl.ds(gb, L)], ds_v, dd_v, agg_v, h_v,
                    lambda e, r16: r16[e])

    _add_rows(agg_v, agg_sh, iidx, ND2P)
    plsc.subcore_barrier()

    @pl.when(sid == 0)
    def _():
        pltpu.sync_copy(agg_sh, aggp.at[cid])


@jax.jit
def _agg2_call(row, col, w, dinv, aggp1, b1, zsmall):
    return pl.kernel(
        _agg2_body,
        out_type=jax.ShapeDtypeStruct((NC, ND2P, F_HID), jnp.float32),
        mesh=_mesh,
        compiler_params=_sc_params,
        scratch_types=[
            pltpu.VMEM((ND2P, F_HID), jnp.float32),
            pltpu.VMEM((N_DST1,), jnp.float32),
            pltpu.VMEM((N_DST2,), jnp.float32),
            pltpu.VMEM((F_HID,), jnp.float32),
            pltpu.VMEM((ND1P // NS, F_HID), jnp.float32),
            pltpu.VMEM((ND1P // NS, F_HID), jnp.float32),
            pltpu.VMEM((E2T,), jnp.int32),
            pltpu.VMEM((E2T,), jnp.int32),
            pltpu.VMEM((E2T,), jnp.float32),
            pltpu.VMEM((ND1P, F_HID), jnp.float32),
            pltpu.VMEM((1, 128), jnp.int32),
            pltpu.VMEM_SHARED((ND1P, F_HID), jnp.float32),
            pltpu.VMEM_SHARED((ND2P, F_HID), jnp.float32),
        ],
    )(row, col, w, dinv, aggp1, b1, zsmall)


def _tc_prep_body(x_ref, w1_ref, degp_ref, xlin_ref, dinv_ref):
    xlin_ref[...] = jnp.dot(x_ref[...], w1_ref[...],
                            preferred_element_type=jnp.float32)
    deg = degp_ref[0] + degp_ref[1]
    dinv_ref[...] = jnp.where(deg > 0.0, lax.rsqrt(deg), 0.0)


@jax.jit
def _tc_prep(x, W1, degp):
    return pl.pallas_call(
        _tc_prep_body,
        out_shape=[
            jax.ShapeDtypeStruct((N_SRC0, F_HID), jnp.float32),
            jax.ShapeDtypeStruct((DEG_ROWS, 128), jnp.float32),
        ],
    )(x, W1, degp)


def _tc_final_body(aggp_ref, w2_ref, b2_ref, out_ref):
    agg = (aggp_ref[0] + aggp_ref[1])[:N_DST2]
    o = jnp.dot(agg, w2_ref[...], preferred_element_type=jnp.float32)
    o = o + b2_ref[...]
    z = o - jnp.max(o, axis=1, keepdims=True)
    out_ref[...] = z - jnp.log(jnp.sum(jnp.exp(z), axis=1, keepdims=True))


@jax.jit
def _tc_final(aggp, W2, b2):
    return pl.pallas_call(
        _tc_final_body,
        out_shape=jax.ShapeDtypeStruct((N_DST2, F_OUT), jnp.float32),
    )(aggp, W2, b2)


def kernel(x, edge_index1, e_id1, edge_index2, e_id2, edge_weight,
           W1, b1, W2, b2):
    i32 = jnp.int32
    loops1 = jnp.arange(N_DST1, dtype=i32)
    loops2 = jnp.arange(N_DST2, dtype=i32)
    pad1 = E1P - E1 - N_DST1
    pad2 = E2P - E2 - N_DST2
    z1 = jnp.zeros((pad1,), i32)
    z2 = jnp.zeros((pad2,), i32)
    r1 = jnp.concatenate([edge_index1[0], loops1, z1])
    c1 = jnp.concatenate([edge_index1[1], loops1, z1])
    i1 = jnp.concatenate([e_id1, jnp.full((N_DST1,), ONE_ID, i32),
                          jnp.full((pad1,), ZERO_ID, i32)])
    r2 = jnp.concatenate([edge_index2[0], loops2, z2])
    c2 = jnp.concatenate([edge_index2[1], loops2, z2])
    i2 = jnp.concatenate([e_id2, jnp.full((N_DST2,), ONE_ID, i32),
                          jnp.full((pad2,), ZERO_ID, i32)])
    ew = jnp.concatenate([
        edge_weight,
        jnp.array([0.0, 1.0], jnp.float32),
        jnp.zeros((EW_PAD - E_TOT - 2,), jnp.float32),
    ])
    zerd = jnp.zeros((DEG_ROWS, 128), jnp.float32)
    zsmall = jnp.zeros((128, F_HID), jnp.float32)

    w1, w2, degp = _deg_call(r1, c1, i1, r2, c2, i2, ew, zerd)
    xlin, dinv2d = _tc_prep(x, W1, degp)
    dinv = dinv2d.reshape(DEG_TOT)
    rc1 = r1 * 16384 + c1
    aggp1 = _agg1_call(rc1, w1, dinv, xlin, zsmall)
    aggp2 = _agg2_call(r2, c2, w2, dinv, aggp1, b1, zsmall)
    return _tc_final(aggp2, W2, b2.reshape(1, F_OUT))
